# speedup vs baseline: 18.5280x; 18.5280x over previous
import functools

import jax
import jax.numpy as jnp
from jax.experimental import pallas as pl
from jax.experimental.pallas import tpu as pltpu

_B, _N, _M = 8, 4096, 1024
_NCOMP = 8
_D = 16
_CKF = 256
_CUF = 128
_EPS_BN = 1e-5
_NTOT = float(_B * _N)

_BLK_N1 = 512
_BLK_N2 = 1024


def _k1_body(fut_ref, fk_ref, kf_ref, uf_ref, w1_ref,
             y1_ref, s1_ref, q1_ref):
    b = pl.program_id(0)
    i = pl.program_id(1)

    fu = fut_ref[0]
    fk = fk_ref[0]
    ssu = jnp.sum(fu * fu, axis=0, keepdims=True)
    ssk = jnp.sum(fk * fk, axis=1, keepdims=True)
    cross = jax.lax.dot_general(fk, fu, (((1,), (0,)), ((), ())),
                                preferred_element_type=jnp.float32)
    d2 = ssk + ssu - 2.0 * cross

    row = jax.lax.broadcasted_iota(jnp.int32, (_M, _BLK_N1), 0)
    inf = jnp.float32(jnp.inf)
    d = d2
    recips = []
    sels = []
    for _ in range(3):
        vk = jnp.min(d, axis=0, keepdims=True)
        ik = jnp.min(jnp.where(d == vk, row, _M), axis=0, keepdims=True)
        sel = row == ik
        d = jnp.where(sel, inf, d)
        dist = jnp.sqrt(jnp.maximum(vk, 0.0))
        recips.append(1.0 / (dist + 1e-8))
        sels.append(sel)
    norm = recips[0] + recips[1] + recips[2]
    st = jnp.zeros((_M, _BLK_N1), jnp.float32)
    for k in range(3):
        st = jnp.where(sels[k], recips[k] / norm, st)

    interp = jax.lax.dot_general(kf_ref[0], st, (((1,), (0,)), ((), ())),
                                 preferred_element_type=jnp.float32)
    x = jnp.concatenate([interp, uf_ref[0]], axis=0)
    y1 = jax.lax.dot_general(w1_ref[...], x, (((1,), (0,)), ((), ())),
                             preferred_element_type=jnp.float32)

    @pl.when(jnp.logical_and(b == 0, i == 0))
    def _():
        s1_ref[...] = jnp.zeros_like(s1_ref)
        q1_ref[...] = jnp.zeros_like(q1_ref)

    s1_ref[...] += jnp.sum(y1, axis=1, keepdims=True)
    q1_ref[...] += jnp.sum(y1 * y1, axis=1, keepdims=True)
    y1_ref[0] = y1


def _bn_scale_shift(s_ref, q_ref, g_ref, b_ref):
    mean = s_ref[...] / _NTOT
    var = q_ref[...] / _NTOT - mean * mean
    scale = g_ref[...] / jnp.sqrt(var + _EPS_BN)
    shift = b_ref[...] - mean * scale
    return scale, shift


def _k2_body(y1_ref, s1_ref, q1_ref, g1_ref, b1_ref, w2_ref,
             y2_ref, s2_ref, q2_ref):
    b = pl.program_id(0)
    i = pl.program_id(1)
    scale, shift = _bn_scale_shift(s1_ref, q1_ref, g1_ref, b1_ref)
    h = jnp.maximum(y1_ref[0] * scale + shift, 0.0)
    y2 = jax.lax.dot_general(w2_ref[...], h, (((1,), (0,)), ((), ())),
                             preferred_element_type=jnp.float32)

    @pl.when(jnp.logical_and(b == 0, i == 0))
    def _():
        s2_ref[...] = jnp.zeros_like(s2_ref)
        q2_ref[...] = jnp.zeros_like(q2_ref)

    s2_ref[...] += jnp.sum(y2, axis=1, keepdims=True)
    q2_ref[...] += jnp.sum(y2 * y2, axis=1, keepdims=True)
    y2_ref[0] = y2


def _k3_body(y2_ref, s2_ref, q2_ref, g2_ref, b2_ref, h2_ref):
    scale, shift = _bn_scale_shift(s2_ref, q2_ref, g2_ref, b2_ref)
    h2_ref[0] = jnp.maximum(y2_ref[0] * scale + shift, 0.0)


@jax.jit
def kernel(unknown, known, unknown_feats, known_feats, W1, g1, b1, W2, g2, b2):
    unk_comp = unknown_feats[:, :_NCOMP, :]
    uf = unknown_feats[:, _NCOMP:, :]
    kf = known_feats[:, _NCOMP:, :]
    kc = jnp.transpose(known_feats[:, :_NCOMP, :], (0, 2, 1))

    fut = jnp.concatenate(
        [jnp.transpose(unknown, (0, 2, 1)), unk_comp,
         jnp.zeros((_B, _D - 3 - _NCOMP, _N), jnp.float32)], axis=1)
    fk = jnp.concatenate(
        [known, kc, jnp.zeros((_B, _M, _D - 3 - _NCOMP), jnp.float32)],
        axis=-1)

    cvec = pl.BlockSpec((_CKF, 1), lambda b, i: (0, 0))

    y1, s1, q1 = pl.pallas_call(
        _k1_body,
        grid=(_B, _N // _BLK_N1),
        in_specs=[
            pl.BlockSpec((1, _D, _BLK_N1), lambda b, i: (b, 0, i)),
            pl.BlockSpec((1, _M, _D), lambda b, i: (b, 0, 0)),
            pl.BlockSpec((1, _CKF, _M), lambda b, i: (b, 0, 0)),
            pl.BlockSpec((1, _CUF, _BLK_N1), lambda b, i: (b, 0, i)),
            pl.BlockSpec((_CKF, _CKF + _CUF), lambda b, i: (0, 0)),
        ],
        out_specs=[
            pl.BlockSpec((1, _CKF, _BLK_N1), lambda b, i: (b, 0, i)),
            cvec, cvec,
        ],
        out_shape=[
            jax.ShapeDtypeStruct((_B, _CKF, _N), jnp.float32),
            jax.ShapeDtypeStruct((_CKF, 1), jnp.float32),
            jax.ShapeDtypeStruct((_CKF, 1), jnp.float32),
        ],
    )(fut, fk, kf, uf, W1)

    g1c = g1.reshape(_CKF, 1)
    b1c = b1.reshape(_CKF, 1)
    g2c = g2.reshape(_CKF, 1)
    b2c = b2.reshape(_CKF, 1)

    y2, s2, q2 = pl.pallas_call(
        _k2_body,
        grid=(_B, _N // _BLK_N2),
        in_specs=[
            pl.BlockSpec((1, _CKF, _BLK_N2), lambda b, i: (b, 0, i)),
            cvec, cvec, cvec, cvec,
            pl.BlockSpec((_CKF, _CKF), lambda b, i: (0, 0)),
        ],
        out_specs=[
            pl.BlockSpec((1, _CKF, _BLK_N2), lambda b, i: (b, 0, i)),
            cvec, cvec,
        ],
        out_shape=[
            jax.ShapeDtypeStruct((_B, _CKF, _N), jnp.float32),
            jax.ShapeDtypeStruct((_CKF, 1), jnp.float32),
            jax.ShapeDtypeStruct((_CKF, 1), jnp.float32),
        ],
    )(y1, s1, q1, g1c, b1c, W2)

    h2 = pl.pallas_call(
        _k3_body,
        grid=(_B, _N // _BLK_N2),
        in_specs=[
            pl.BlockSpec((1, _CKF, _BLK_N2), lambda b, i: (b, 0, i)),
            cvec, cvec, cvec, cvec,
        ],
        out_specs=pl.BlockSpec((1, _CKF, _BLK_N2), lambda b, i: (b, 0, i)),
        out_shape=jax.ShapeDtypeStruct((_B, _CKF, _N), jnp.float32),
    )(y2, s2, q2, g2c, b2c)

    return jnp.concatenate([unk_comp, h2], axis=1)


# speedup vs baseline: 23.3688x; 1.2613x over previous
import functools

import jax
import jax.numpy as jnp
from jax.experimental import pallas as pl
from jax.experimental.pallas import tpu as pltpu

_B, _N, _M = 8, 4096, 1024
_NCOMP = 8
_D = 16
_CKF = 256
_CUF = 128
_EPS_BN = 1e-5
_NTOT = float(_B * _N)

_BLK_N1 = 512
_BLK_N2 = 1024


def _k1_body(fut_ref, fk_ref, kf_ref, uf_ref, w1_ref,
             y1_ref, s1_ref, q1_ref, d2_ref):
    b = pl.program_id(0)
    i = pl.program_id(1)

    fu = fut_ref[0]
    fk = fk_ref[0]
    ssu = jnp.sum(fu * fu, axis=0, keepdims=True)
    ssk = jnp.sum(fk * fk, axis=1, keepdims=True)
    cross2 = jax.lax.dot_general(fk, -2.0 * fu, (((1,), (0,)), ((), ())),
                                 preferred_element_type=jnp.float32)
    d2_ref[...] = ssk + cross2

    inf = jnp.float32(jnp.inf)
    m_init = jnp.full((8, _BLK_N1), inf, jnp.float32)

    def _insert(m, v):
        m1, m2, m3 = m
        t1 = jnp.maximum(m1, v)
        m1 = jnp.minimum(m1, v)
        t2 = jnp.maximum(m2, t1)
        m2 = jnp.minimum(m2, t1)
        m3 = jnp.minimum(m3, t2)
        return m1, m2, m3

    def _chunk(c, m):
        base = c * 32
        for j in range(4):
            m = _insert(m, d2_ref[pl.ds(base + 8 * j, 8), :])
        return m

    m1, m2, m3 = jax.lax.fori_loop(0, _M // 32, _chunk,
                                   (m_init, m_init, m_init), unroll=2)

    cand = jnp.concatenate([m1, m2, m3], axis=0)
    row24 = jax.lax.broadcasted_iota(jnp.int32, (24, _BLK_N1), 0)
    vs = []
    for _ in range(3):
        vk = jnp.min(cand, axis=0, keepdims=True)
        ik = jnp.min(jnp.where(cand == vk, row24, 24), axis=0, keepdims=True)
        cand = jnp.where(row24 == ik, inf, cand)
        vs.append(vk)

    recips = [1.0 / (jnp.sqrt(jnp.maximum(v + ssu, 0.0)) + 1e-8) for v in vs]
    norm = recips[0] + recips[1] + recips[2]
    w = [r / norm for r in recips]

    d2 = d2_ref[...]
    st = jnp.where(d2 == vs[0], w[0],
                   jnp.where(d2 == vs[1], w[1],
                             jnp.where(d2 == vs[2], w[2], 0.0)))

    interp = jax.lax.dot_general(kf_ref[0], st, (((1,), (0,)), ((), ())),
                                 preferred_element_type=jnp.float32)
    x = jnp.concatenate([interp, uf_ref[0]], axis=0)
    y1 = jax.lax.dot_general(w1_ref[...], x, (((1,), (0,)), ((), ())),
                             preferred_element_type=jnp.float32)

    @pl.when(jnp.logical_and(b == 0, i == 0))
    def _():
        s1_ref[...] = jnp.zeros_like(s1_ref)
        q1_ref[...] = jnp.zeros_like(q1_ref)

    s1_ref[...] += jnp.sum(y1, axis=1, keepdims=True)
    q1_ref[...] += jnp.sum(y1 * y1, axis=1, keepdims=True)
    y1_ref[0] = y1


def _bn_scale_shift(s_ref, q_ref, g_ref, b_ref):
    mean = s_ref[...] / _NTOT
    var = q_ref[...] / _NTOT - mean * mean
    scale = g_ref[...] / jnp.sqrt(var + _EPS_BN)
    shift = b_ref[...] - mean * scale
    return scale, shift


def _k2_body(y1_ref, s1_ref, q1_ref, g1_ref, b1_ref, w2_ref,
             y2_ref, s2_ref, q2_ref):
    b = pl.program_id(0)
    i = pl.program_id(1)
    scale, shift = _bn_scale_shift(s1_ref, q1_ref, g1_ref, b1_ref)
    h = jnp.maximum(y1_ref[0] * scale + shift, 0.0)
    y2 = jax.lax.dot_general(w2_ref[...], h, (((1,), (0,)), ((), ())),
                             preferred_element_type=jnp.float32)

    @pl.when(jnp.logical_and(b == 0, i == 0))
    def _():
        s2_ref[...] = jnp.zeros_like(s2_ref)
        q2_ref[...] = jnp.zeros_like(q2_ref)

    s2_ref[...] += jnp.sum(y2, axis=1, keepdims=True)
    q2_ref[...] += jnp.sum(y2 * y2, axis=1, keepdims=True)
    y2_ref[0] = y2


def _k3_body(y2_ref, s2_ref, q2_ref, g2_ref, b2_ref, h2_ref):
    scale, shift = _bn_scale_shift(s2_ref, q2_ref, g2_ref, b2_ref)
    h2_ref[0] = jnp.maximum(y2_ref[0] * scale + shift, 0.0)


@jax.jit
def kernel(unknown, known, unknown_feats, known_feats, W1, g1, b1, W2, g2, b2):
    unk_comp = unknown_feats[:, :_NCOMP, :]
    uf = unknown_feats[:, _NCOMP:, :]
    kf = known_feats[:, _NCOMP:, :]
    kc = jnp.transpose(known_feats[:, :_NCOMP, :], (0, 2, 1))

    fut = jnp.concatenate(
        [jnp.transpose(unknown, (0, 2, 1)), unk_comp,
         jnp.zeros((_B, _D - 3 - _NCOMP, _N), jnp.float32)], axis=1)
    fk = jnp.concatenate(
        [known, kc, jnp.zeros((_B, _M, _D - 3 - _NCOMP), jnp.float32)],
        axis=-1)

    cvec = pl.BlockSpec((_CKF, 1), lambda b, i: (0, 0))

    y1, s1, q1 = pl.pallas_call(
        _k1_body,
        grid=(_B, _N // _BLK_N1),
        in_specs=[
            pl.BlockSpec((1, _D, _BLK_N1), lambda b, i: (b, 0, i)),
            pl.BlockSpec((1, _M, _D), lambda b, i: (b, 0, 0)),
            pl.BlockSpec((1, _CKF, _M), lambda b, i: (b, 0, 0)),
            pl.BlockSpec((1, _CUF, _BLK_N1), lambda b, i: (b, 0, i)),
            pl.BlockSpec((_CKF, _CKF + _CUF), lambda b, i: (0, 0)),
        ],
        out_specs=[
            pl.BlockSpec((1, _CKF, _BLK_N1), lambda b, i: (b, 0, i)),
            cvec, cvec,
        ],
        out_shape=[
            jax.ShapeDtypeStruct((_B, _CKF, _N), jnp.float32),
            jax.ShapeDtypeStruct((_CKF, 1), jnp.float32),
            jax.ShapeDtypeStruct((_CKF, 1), jnp.float32),
        ],
        scratch_shapes=[pltpu.VMEM((_M, _BLK_N1), jnp.float32)],
    )(fut, fk, kf, uf, W1)

    g1c = g1.reshape(_CKF, 1)
    b1c = b1.reshape(_CKF, 1)
    g2c = g2.reshape(_CKF, 1)
    b2c = b2.reshape(_CKF, 1)

    y2, s2, q2 = pl.pallas_call(
        _k2_body,
        grid=(_B, _N // _BLK_N2),
        in_specs=[
            pl.BlockSpec((1, _CKF, _BLK_N2), lambda b, i: (b, 0, i)),
            cvec, cvec, cvec, cvec,
            pl.BlockSpec((_CKF, _CKF), lambda b, i: (0, 0)),
        ],
        out_specs=[
            pl.BlockSpec((1, _CKF, _BLK_N2), lambda b, i: (b, 0, i)),
            cvec, cvec,
        ],
        out_shape=[
            jax.ShapeDtypeStruct((_B, _CKF, _N), jnp.float32),
            jax.ShapeDtypeStruct((_CKF, 1), jnp.float32),
            jax.ShapeDtypeStruct((_CKF, 1), jnp.float32),
        ],
    )(y1, s1, q1, g1c, b1c, W2)

    h2 = pl.pallas_call(
        _k3_body,
        grid=(_B, _N // _BLK_N2),
        in_specs=[
            pl.BlockSpec((1, _CKF, _BLK_N2), lambda b, i: (b, 0, i)),
            cvec, cvec, cvec, cvec,
        ],
        out_specs=pl.BlockSpec((1, _CKF, _BLK_N2), lambda b, i: (b, 0, i)),
        out_shape=jax.ShapeDtypeStruct((_B, _CKF, _N), jnp.float32),
    )(y2, s2, q2, g2c, b2c)

    return jnp.concatenate([unk_comp, h2], axis=1)
